# baseline (speedup 1.0000x reference)
import functools
import math

import jax
import jax.numpy as jnp
from jax.experimental import pallas as pl
from jax.experimental.pallas import tpu as pltpu

_INV_SQRT2 = 1.0 / math.sqrt(2.0)


def _router_block(nsplit, *refs):
    x_refs = refs[:nsplit]
    w1_ref, b1_ref, w2_ref, b2_ref = refs[nsplit:nsplit + 4]
    packet_refs = refs[nsplit + 4:nsplit + 4 + nsplit]
    prob_refs = refs[nsplit + 4 + nsplit:]
    w1 = w1_ref[...]
    w2 = w2_ref[...]
    b1 = b1_ref[...]
    b2 = b2_ref[...]
    for s in range(nsplit):
        h = jnp.dot(x_refs[s][...], w1, preferred_element_type=jnp.float32)
        h = h + b1
        h = 0.5 * h * (1.0 + jax.lax.erf(h * _INV_SQRT2))
        logits = jnp.dot(h, w2, preferred_element_type=jnp.float32)
        logits = logits + b2
        packet_refs[s][...] = jnp.argmax(
            logits, axis=-1, keepdims=True).astype(jnp.int32)
        m = jnp.max(logits, axis=-1, keepdims=True)
        e = jnp.exp(logits - m)
        prob_refs[s][...] = e / jnp.sum(e, axis=-1, keepdims=True)


@functools.partial(jax.jit, static_argnames=("block_n", "nsplit"))
def kernel(x, W1, b1, W2, b2, block_n: int = 2048, nsplit: int = 2):
    n, d = x.shape
    h_dim = W1.shape[1]
    p = W2.shape[1]
    seg = n // nsplit
    blocks_per_seg = seg // block_n
    grid = (blocks_per_seg,)

    def x_map(i, s=0):
        return (s * blocks_per_seg + i, 0)

    in_specs = [
        pl.BlockSpec((block_n, d), functools.partial(x_map, s=s))
        for s in range(nsplit)
    ] + [
        pl.BlockSpec((d, h_dim), lambda i: (0, 0)),
        pl.BlockSpec((h_dim,), lambda i: (0,)),
        pl.BlockSpec((h_dim, p), lambda i: (0, 0)),
        pl.BlockSpec((p,), lambda i: (0,)),
    ]
    out_specs = (
        [pl.BlockSpec((block_n, 1), lambda i: (i, 0)) for _ in range(nsplit)]
        + [pl.BlockSpec((block_n, p), lambda i: (i, 0)) for _ in range(nsplit)]
    )
    out_shape = (
        [jax.ShapeDtypeStruct((seg, 1), jnp.int32) for _ in range(nsplit)]
        + [jax.ShapeDtypeStruct((seg, p), jnp.float32) for _ in range(nsplit)]
    )
    outs = pl.pallas_call(
        functools.partial(_router_block, nsplit),
        grid=grid,
        in_specs=in_specs,
        out_specs=out_specs,
        out_shape=out_shape,
        compiler_params=pltpu.CompilerParams(
            dimension_semantics=("arbitrary",),
        ),
    )(*([x] * nsplit), W1, b1, W2, b2)
    packets = jnp.concatenate([o.reshape(seg) for o in outs[:nsplit]])
    probs = jnp.concatenate(outs[nsplit:], axis=0)
    return packets, probs


# speedup vs baseline: 1.0417x; 1.0417x over previous
import functools
import math

import jax
import jax.numpy as jnp
from jax.experimental import pallas as pl
from jax.experimental.pallas import tpu as pltpu

_INV_SQRT2 = 1.0 / math.sqrt(2.0)


def _router_block(nsplit, *refs):
    x_refs = refs[:nsplit]
    w1_ref, b1_ref, w2_ref, b2_ref = refs[nsplit:nsplit + 4]
    packet_refs = refs[nsplit + 4:nsplit + 4 + nsplit]
    prob_refs = refs[nsplit + 4 + nsplit:]
    w1 = w1_ref[...]
    w2 = w2_ref[...]
    b1 = b1_ref[...]
    b2 = b2_ref[...]
    for s in range(nsplit):
        h = jnp.dot(x_refs[s][...], w1, preferred_element_type=jnp.float32)
        h = h + b1
        h = 0.5 * h * (1.0 + jax.lax.erf(h * _INV_SQRT2))
        logits = jnp.dot(h, w2, preferred_element_type=jnp.float32)
        logits = logits + b2
        packet_refs[s][...] = jnp.argmax(
            logits, axis=-1, keepdims=True).astype(jnp.int32)
        m = jnp.max(logits, axis=-1, keepdims=True)
        e = jnp.exp(logits - m)
        prob_refs[s][...] = e / jnp.sum(e, axis=-1, keepdims=True)


@functools.partial(jax.jit, static_argnames=("block_n", "nsplit"))
def kernel(x, W1, b1, W2, b2, block_n: int = 2048, nsplit: int = 1):
    n, d = x.shape
    h_dim = W1.shape[1]
    p = W2.shape[1]
    seg = n // nsplit
    blocks_per_seg = seg // block_n
    grid = (blocks_per_seg,)

    def x_map(i, s=0):
        return (s * blocks_per_seg + i, 0)

    in_specs = [
        pl.BlockSpec((block_n, d), functools.partial(x_map, s=s))
        for s in range(nsplit)
    ] + [
        pl.BlockSpec((d, h_dim), lambda i: (0, 0)),
        pl.BlockSpec((h_dim,), lambda i: (0,)),
        pl.BlockSpec((h_dim, p), lambda i: (0, 0)),
        pl.BlockSpec((p,), lambda i: (0,)),
    ]
    out_specs = (
        [pl.BlockSpec((block_n, 1), lambda i: (i, 0)) for _ in range(nsplit)]
        + [pl.BlockSpec((block_n, p), lambda i: (i, 0)) for _ in range(nsplit)]
    )
    out_shape = (
        [jax.ShapeDtypeStruct((seg, 1), jnp.int32) for _ in range(nsplit)]
        + [jax.ShapeDtypeStruct((seg, p), jnp.float32) for _ in range(nsplit)]
    )
    outs = pl.pallas_call(
        functools.partial(_router_block, nsplit),
        grid=grid,
        in_specs=in_specs,
        out_specs=out_specs,
        out_shape=out_shape,
        compiler_params=pltpu.CompilerParams(
            dimension_semantics=("parallel",),
        ),
    )(*([x] * nsplit), W1, b1, W2, b2)
    packets = jnp.concatenate([o.reshape(seg) for o in outs[:nsplit]])
    probs = jnp.concatenate(outs[nsplit:], axis=0)
    return packets, probs
